# baseline (speedup 1.0000x reference)
import functools

import numpy as np
import jax
import jax.numpy as jnp
from jax import lax
from jax.experimental import pallas as pl
from jax.experimental.pallas import tpu as pltpu
from jax.experimental.pallas import tpu_sc as plsc

NUM_LEVELS = 16
F = 2
T = 2 ** 19
MASK = T - 1
_growth = np.exp((np.log(256.0) - np.log(16.0)) / (NUM_LEVELS - 1))
_SCALINGS = np.floor(16.0 * _growth ** np.arange(NUM_LEVELS)).astype(np.float32)
_P = [1, -1640531535, 805459861, -620313867]

N = 65536
NW = 32
PW = N // NW
NGROUP = PW // 16
NB = 4


def _make_kernel():
    mesh = plsc.VectorSubcoreMesh(
        core_axis_name="c", subcore_axis_name="s", num_cores=2, num_subcores=16
    )

    scratch = [
        pltpu.VMEM((4, PW), jnp.float32),
        pltpu.VMEM((16, 16), jnp.float32),
        pltpu.VMEM((F * NUM_LEVELS, 128), jnp.float32),
    ]
    scratch += [pltpu.VMEM((4096,), jnp.int32) for _ in range(NB)]
    scratch += [pltpu.VMEM((4096,), jnp.float32) for _ in range(NB)]
    scratch += [pltpu.VMEM((4096,), jnp.int32) for _ in range(NB)]
    scratch += [pltpu.SemaphoreType.DMA for _ in range(NB)]

    @functools.partial(
        pl.kernel,
        out_type=jax.ShapeDtypeStruct((F * NUM_LEVELS, N), jnp.float32),
        mesh=mesh,
        scratch_types=scratch,
    )
    def encode(xt_hbm, tab_hbm, scal_hbm, out_hbm, x_v, scal_v, out_s, *bufs):
        idx = bufs[0:NB]
        w = bufs[NB:2 * NB]
        rows = bufs[2 * NB:3 * NB]
        sem = bufs[3 * NB:4 * NB]

        cid = lax.axis_index("c")
        sid = lax.axis_index("s")
        wid = sid * 2 + cid
        base = pl.multiple_of(wid * PW, PW)
        pltpu.sync_copy(xt_hbm.at[:, pl.ds(base, PW)], x_v)
        pltpu.sync_copy(scal_hbm, scal_v)

        def phase_a(g, b):
            xg = [x_v[d, pl.ds(g * 16, 16)] for d in range(4)]

            @pl.loop(0, NUM_LEVELS)
            def _lvl(l):
                s = scal_v[l, :]
                lofs = lax.broadcast(l * T, (16,))
                m0, m1, off, om = [], [], [], []
                for d in range(4):
                    scaled = xg[d] * s
                    sfi = scaled.astype(jnp.int32)
                    sf = sfi.astype(jnp.float32)
                    off_d = scaled - sf
                    om_d = 1.0 - off_d
                    m0_d = sfi if d == 0 else sfi * _P[d]
                    m1_d = m0_d + _P[d]
                    m0.append(m0_d); m1.append(m1_d)
                    off.append(off_d); om.append(om_d)
                a01 = [m0[0] ^ m0[1], m1[0] ^ m0[1], m0[0] ^ m1[1], m1[0] ^ m1[1]]
                w01 = [om[0] * om[1], off[0] * om[1], om[0] * off[1], off[0] * off[1]]
                a23 = [m0[2] ^ m0[3], m1[2] ^ m0[3], m0[2] ^ m1[3], m1[2] ^ m1[3]]
                w23 = [om[2] * om[3], off[2] * om[3], om[2] * off[3], off[2] * off[3]]
                for c in range(16):
                    idxv = ((a01[c & 3] ^ a23[(c >> 2) & 3]) & MASK) + lofs
                    j = l * 16 + c
                    idx[b][pl.ds(j * 16, 16)] = idxv
                    w[b][pl.ds(j * 16, 16)] = w01[c & 3] * w23[(c >> 2) & 3]

        def fire(b):
            for k in range(4):
                pltpu.async_copy(
                    tab_hbm.at[idx[b].at[pl.ds(k * 1024, 1024)]],
                    rows[b].at[pl.ds(k * 1024, 1024)], sem[b])

        def drain(b):
            for k in range(4):
                pltpu.make_async_copy(
                    tab_hbm.at[idx[b].at[pl.ds(k * 1024, 1024)]],
                    rows[b].at[pl.ds(k * 1024, 1024)], sem[b]
                ).wait()

        def phase_c(g, b):
            gc = (g & 7) * 16

            @pl.loop(0, NUM_LEVELS)
            def _lvl(l):
                acc0 = jnp.zeros((16,), jnp.float32)
                acc1 = jnp.zeros((16,), jnp.float32)
                for c in range(16):
                    j = l * 16 + c
                    v = rows[b][pl.ds(j * 16, 16)]
                    v0 = lax.bitcast_convert_type(v << 16, jnp.float32)
                    v1 = lax.bitcast_convert_type(v & (-65536), jnp.float32)
                    wv = w[b][pl.ds(j * 16, 16)]
                    acc0 = acc0 + wv * v0
                    acc1 = acc1 + wv * v1
                out_s[l * 2, pl.ds(gc, 16)] = acc0
                out_s[l * 2 + 1, pl.ds(gc, 16)] = acc1

            @pl.when((g & 7) == 7)
            def _flush():
                pltpu.sync_copy(
                    out_s,
                    out_hbm.at[:, pl.ds(pl.multiple_of(base + (g - 7) * 16, 128),
                                        128)])

        for b in range(NB - 1):
            phase_a(b, b)
            fire(b)

        @pl.loop(0, NGROUP // NB - 1)
        def _grp(k):
            g0 = k * NB
            for b in range(NB):
                phase_a(g0 + b + (NB - 1), (b + NB - 1) % NB)
                fire((b + NB - 1) % NB)
                drain(b)
                phase_c(g0 + b, b)

        g0 = NGROUP - NB
        phase_a(NGROUP - 1, NB - 1)
        fire(NB - 1)
        for b in range(NB):
            drain(b)
            phase_c(g0 + b, b)

    return encode


_encode = _make_kernel()


def kernel(xyzt, hash_table):
    with jax.enable_x64(False):
        xt = xyzt.astype(jnp.float32).T
        scal = jnp.broadcast_to(jnp.asarray(_SCALINGS)[:, None], (16, 16))
        u = lax.bitcast_convert_type(hash_table.astype(jnp.float32),
                                     jnp.uint32)
        r = (u + 0x7FFF + ((u >> 16) & 1)) >> 16
        packed = jnp.sum(r * jnp.asarray([1, 65536], jnp.uint32)[None, :],
                         axis=1, dtype=jnp.uint32)
        tab_pair = lax.bitcast_convert_type(packed, jnp.int32)
        out3 = _encode(xt, tab_pair, scal)
        return out3.T


# speedup vs baseline: 1.6390x; 1.6390x over previous
import functools

import numpy as np
import jax
import jax.numpy as jnp
from jax import lax
from jax.experimental import pallas as pl
from jax.experimental.pallas import tpu as pltpu
from jax.experimental.pallas import tpu_sc as plsc

NUM_LEVELS = 16
F = 2
T = 2 ** 19
MASK = T - 1
_growth = np.exp((np.log(256.0) - np.log(16.0)) / (NUM_LEVELS - 1))
_SCALINGS = np.floor(16.0 * _growth ** np.arange(NUM_LEVELS)).astype(np.float32)
_P = [1, -1640531535, 805459861, -620313867]

N = 65536
NW = 32
PW = N // NW
NQ = PW // 64
TQ = T // 16


def _make_kernel():
    mesh = plsc.VectorSubcoreMesh(
        core_axis_name="c", subcore_axis_name="s", num_cores=2, num_subcores=16
    )

    scratch = [
        pltpu.VMEM((4, PW), jnp.float32),
        pltpu.VMEM((16, 16), jnp.float32),
        pltpu.VMEM((2, PW), jnp.float32),
        pltpu.VMEM((1024,), jnp.int32),
        pltpu.VMEM((1024,), jnp.int32),
        pltpu.VMEM((1024,), jnp.float32),
        pltpu.VMEM((1024,), jnp.float32),
        pltpu.VMEM((1024,), jnp.int32),
        pltpu.VMEM((1024,), jnp.int32),
        pltpu.VMEM_SHARED((T,), jnp.int32),
        pltpu.VMEM_SHARED((T,), jnp.int32),
        pltpu.SemaphoreType.DMA,
        pltpu.SemaphoreType.DMA,
        pltpu.SemaphoreType.DMA,
        pltpu.SemaphoreType.DMA,
    ]

    @functools.partial(
        pl.kernel,
        out_type=jax.ShapeDtypeStruct((NUM_LEVELS, F, N), jnp.float32),
        mesh=mesh,
        scratch_types=scratch,
    )
    def encode(xt_hbm, tab_hbm, scal_hbm, out_hbm,
               x_v, scal_v, out_l, idx_a, idx_b, w_a, w_b, rows_a, rows_b,
               sp_a, sp_b, gsem_a, gsem_b, ssem_a, ssem_b):
        cid = lax.axis_index("c")
        sid = lax.axis_index("s")
        wid = sid * 2 + cid
        base = pl.multiple_of(wid * PW, PW)
        soff = pl.multiple_of(sid * TQ, TQ)
        pltpu.sync_copy(xt_hbm.at[:, pl.ds(base, PW)], x_v)
        pltpu.sync_copy(scal_hbm, scal_v)

        def stage(l, sp, ssem):
            pltpu.async_copy(tab_hbm.at[pl.ds(l * T + soff, TQ)],
                             sp.at[pl.ds(soff, TQ)], ssem)

        def stage_wait(l, sp, ssem):
            pltpu.make_async_copy(tab_hbm.at[pl.ds(l * T + soff, TQ)],
                                  sp.at[pl.ds(soff, TQ)], ssem).wait()
            plsc.subcore_barrier()

        def phase_a(l, q, idx_ref, w_ref):
            s = scal_v[l, :]
            for sub in range(4):
                g = q * 4 + sub
                xg = [x_v[d, pl.ds(g * 16, 16)] for d in range(4)]
                m0, m1, off, om = [], [], [], []
                for d in range(4):
                    scaled = xg[d] * s
                    sfi = scaled.astype(jnp.int32)
                    sf = sfi.astype(jnp.float32)
                    off_d = scaled - sf
                    om_d = 1.0 - off_d
                    m0_d = sfi if d == 0 else sfi * _P[d]
                    m1_d = m0_d + _P[d]
                    m0.append(m0_d); m1.append(m1_d)
                    off.append(off_d); om.append(om_d)
                a01 = [m0[0] ^ m0[1], m1[0] ^ m0[1], m0[0] ^ m1[1], m1[0] ^ m1[1]]
                w01 = [om[0] * om[1], off[0] * om[1], om[0] * off[1], off[0] * off[1]]
                a23 = [m0[2] ^ m0[3], m1[2] ^ m0[3], m0[2] ^ m1[3], m1[2] ^ m1[3]]
                w23 = [om[2] * om[3], off[2] * om[3], om[2] * off[3], off[2] * off[3]]
                for c in range(16):
                    idxv = (a01[c & 3] ^ a23[(c >> 2) & 3]) & MASK
                    pos = sub * 256 + c * 16
                    idx_ref[pl.ds(pos, 16)] = idxv
                    w_ref[pl.ds(pos, 16)] = w01[c & 3] * w23[(c >> 2) & 3]

        def fire(sp, idx_ref, rows_ref, gsem):
            pltpu.async_copy(sp.at[idx_ref], rows_ref, gsem)

        def drain(sp, idx_ref, rows_ref, gsem):
            pltpu.make_async_copy(sp.at[idx_ref], rows_ref, gsem).wait()

        def phase_c(q, w_ref, rows_ref):
            for sub in range(4):
                g = q * 4 + sub
                acc0 = jnp.zeros((16,), jnp.float32)
                acc1 = jnp.zeros((16,), jnp.float32)
                for c in range(16):
                    pos = sub * 256 + c * 16
                    v = rows_ref[pl.ds(pos, 16)]
                    v0 = lax.bitcast_convert_type(v << 16, jnp.float32)
                    v1 = lax.bitcast_convert_type(v & (-65536), jnp.float32)
                    wv = w_ref[pl.ds(pos, 16)]
                    acc0 = acc0 + wv * v0
                    acc1 = acc1 + wv * v1
                out_l[0, pl.ds(g * 16, 16)] = acc0
                out_l[1, pl.ds(g * 16, 16)] = acc1

        def process_level(l, sp):
            phase_a(l, 0, idx_a, w_a)
            fire(sp, idx_a, rows_a, gsem_a)

            @pl.loop(0, NQ // 2 - 1)
            def _q(k):
                q = k * 2
                phase_a(l, q + 1, idx_b, w_b)
                fire(sp, idx_b, rows_b, gsem_b)
                drain(sp, idx_a, rows_a, gsem_a)
                phase_c(q, w_a, rows_a)
                phase_a(l, q + 2, idx_a, w_a)
                fire(sp, idx_a, rows_a, gsem_a)
                drain(sp, idx_b, rows_b, gsem_b)
                phase_c(q + 1, w_b, rows_b)

            phase_a(l, NQ - 1, idx_b, w_b)
            fire(sp, idx_b, rows_b, gsem_b)
            drain(sp, idx_a, rows_a, gsem_a)
            phase_c(NQ - 2, w_a, rows_a)
            drain(sp, idx_b, rows_b, gsem_b)
            phase_c(NQ - 1, w_b, rows_b)
            pltpu.sync_copy(out_l, out_hbm.at[l, :, pl.ds(base, PW)])
            plsc.subcore_barrier()

        stage(0, sp_a, ssem_a)

        @pl.loop(0, NUM_LEVELS // 2)
        def _lvl2(li):
            l = li * 2
            stage(l + 1, sp_b, ssem_b)
            stage_wait(l, sp_a, ssem_a)
            process_level(l, sp_a)

            @pl.when(li < NUM_LEVELS // 2 - 1)
            def _next():
                stage(l + 2, sp_a, ssem_a)

            stage_wait(l + 1, sp_b, ssem_b)
            process_level(l + 1, sp_b)

    return encode


_encode = _make_kernel()


def kernel(xyzt, hash_table):
    with jax.enable_x64(False):
        xt = xyzt.astype(jnp.float32).T
        scal = jnp.broadcast_to(jnp.asarray(_SCALINGS)[:, None], (16, 16))
        u = lax.bitcast_convert_type(hash_table.astype(jnp.float32),
                                     jnp.uint32)
        r = (u + 0x7FFF + ((u >> 16) & 1)) >> 16
        packed = jnp.sum(r * jnp.asarray([1, 65536], jnp.uint32)[None, :],
                         axis=1, dtype=jnp.uint32)
        tab_pair = lax.bitcast_convert_type(packed, jnp.int32)
        out4 = _encode(xt, tab_pair, scal)
        return out4.transpose(2, 0, 1).reshape(N, NUM_LEVELS * F)
